# baseline (speedup 1.0000x reference)
import functools

import jax
import jax.numpy as jnp
from jax import lax
from jax.experimental import pallas as pl
from jax.experimental.pallas import tpu as pltpu


def _cell(gates, c, n):
    i = jax.nn.sigmoid(gates[:, 0:n])
    f = jax.nn.sigmoid(gates[:, n:2 * n])
    g = jnp.tanh(gates[:, 2 * n:3 * n])
    o = jax.nn.sigmoid(gates[:, 3 * n:4 * n])
    c_new = f * c + i * g
    h_new = o * jnp.tanh(c_new)
    return h_new, c_new


def _lstm2_kernel(x_ref, wih1_ref, whh1_ref, wih2_ref, whh2_ref,
                  b1_ref, b2_ref,
                  out_ref,
                  xw1_s, xw2_s, h1b_s, h1c, c1c, h2c, c2c,
                  *, t_blk, batch, hidden, out_dim, nb):
    B, H, O = batch, hidden, out_dim
    i = pl.program_id(1)

    @pl.when(i == 0)
    def _():
        h1c[...] = jnp.zeros_like(h1c)
        c1c[...] = jnp.zeros_like(c1c)
        xw2_s[...] = jnp.zeros_like(xw2_s)
        h2c[...] = jnp.zeros_like(h2c)
        c2c[...] = jnp.zeros_like(c2c)

    @pl.when(i == 1)
    def _():
        h2c[...] = jnp.zeros_like(h2c)
        c2c[...] = jnp.zeros_like(c2c)

    proj = jnp.dot(x_ref[...].reshape(t_blk * B, x_ref.shape[-1]),
                   wih1_ref[...], preferred_element_type=jnp.float32)
    xw1_s[...] = (proj + b1_ref[...]).reshape(t_blk, B, 4 * H)

    whh1 = whh1_ref[...]
    whh2 = whh2_ref[...]

    def step(k, carry):
        h1, c1, h2, c2 = carry
        g1 = jnp.dot(h1, whh1, preferred_element_type=jnp.float32) + xw1_s[k]
        g2 = jnp.dot(h2, whh2, preferred_element_type=jnp.float32) + xw2_s[k]
        h1n, c1n = _cell(g1, c1, H)
        h2n, c2n = _cell(g2, c2, O)
        h1b_s[k] = h1n.astype(jnp.bfloat16)
        out_ref[k] = h2n
        return (h1n.astype(jnp.bfloat16), c1n, h2n.astype(jnp.bfloat16), c2n)

    carry = (h1c[...], c1c[...], h2c[...], c2c[...])
    h1, c1, h2, c2 = lax.fori_loop(0, t_blk, step, carry,
                                   unroll=min(t_blk, 8))
    h1c[...] = h1
    c1c[...] = c1
    h2c[...] = h2
    c2c[...] = c2

    xw2_s[...] = (jnp.dot(h1b_s[...].reshape(t_blk * B, H), wih2_ref[...],
                          preferred_element_type=jnp.float32)
                  + b2_ref[...]).reshape(t_blk, B, 4 * O)


def _stacked_lstm(x, p1, p2, *, t_blk=32):
    S, B, D = x.shape
    H = p1['w_hh'].shape[1]
    O = p2['w_hh'].shape[1]
    f32, bf16 = jnp.float32, jnp.bfloat16

    while S % t_blk:
        t_blk //= 2
    nb = S // t_blk
    ncores = 1
    Bc = B // ncores

    xb = x.astype(bf16)
    wih1 = jnp.asarray(p1['w_ih'].T, bf16)
    whh1 = jnp.asarray(p1['w_hh'].T, bf16)
    wih2 = jnp.asarray(p2['w_ih'].T, bf16)
    whh2 = jnp.asarray(p2['w_hh'].T, bf16)
    b1 = (p1['b_ih'] + p1['b_hh']).reshape(1, 4 * H).astype(f32)
    b2 = (p2['b_ih'] + p2['b_hh']).reshape(1, 4 * O).astype(f32)

    body = functools.partial(_lstm2_kernel, t_blk=t_blk, batch=Bc,
                             hidden=H, out_dim=O, nb=nb)

    x_idx = lambda c, i: (jnp.minimum(i, nb - 1), c, 0)
    o_idx = lambda c, i: (jnp.maximum(i - 1, 0), c, 0)
    const = lambda c, i: (0, 0)

    out = pl.pallas_call(
        body,
        out_shape=jax.ShapeDtypeStruct((S, B, O), f32),
        grid_spec=pltpu.PrefetchScalarGridSpec(
            num_scalar_prefetch=0,
            grid=(ncores, nb + 1),
            in_specs=[
                pl.BlockSpec((t_blk, Bc, D), x_idx),
                pl.BlockSpec((D, 4 * H), const),
                pl.BlockSpec((H, 4 * H), const),
                pl.BlockSpec((H, 4 * O), const),
                pl.BlockSpec((O, 4 * O), const),
                pl.BlockSpec((1, 4 * H), const),
                pl.BlockSpec((1, 4 * O), const),
            ],
            out_specs=pl.BlockSpec((t_blk, Bc, O), o_idx),
            scratch_shapes=[
                pltpu.VMEM((t_blk, Bc, 4 * H), f32),
                pltpu.VMEM((t_blk, Bc, 4 * O), f32),
                pltpu.VMEM((t_blk, Bc, H), bf16),
                pltpu.VMEM((Bc, H), bf16),
                pltpu.VMEM((Bc, H), f32),
                pltpu.VMEM((Bc, O), bf16),
                pltpu.VMEM((Bc, O), f32),
            ],
        ),
        compiler_params=pltpu.CompilerParams(
            dimension_semantics=("parallel", "arbitrary"),
            vmem_limit_bytes=100 * 1024 * 1024),
    )(xb, wih1, whh1, wih2, whh2, b1, b2)
    return out


def kernel(sequence, lstm_w_ih, lstm_w_hh, lstm_b_ih, lstm_b_hh,
           fc_w_ih, fc_w_hh, fc_b_ih, fc_b_hh):
    p1 = dict(w_ih=lstm_w_ih, w_hh=lstm_w_hh, b_ih=lstm_b_ih, b_hh=lstm_b_hh)
    p2 = dict(w_ih=fc_w_ih, w_hh=fc_w_hh, b_ih=fc_b_ih, b_hh=fc_b_hh)
    return _stacked_lstm(sequence, p1, p2)


# speedup vs baseline: 1.0974x; 1.0974x over previous
import functools

import jax
import jax.numpy as jnp
from jax import lax
from jax.experimental import pallas as pl
from jax.experimental.pallas import tpu as pltpu


def _cell(gates, c, n):
    i = jax.nn.sigmoid(gates[:, 0:n])
    f = jax.nn.sigmoid(gates[:, n:2 * n])
    g = jnp.tanh(gates[:, 2 * n:3 * n])
    o = jax.nn.sigmoid(gates[:, 3 * n:4 * n])
    c_new = f * c + i * g
    h_new = o * jnp.tanh(c_new)
    return h_new, c_new


def _lstm2_kernel(x_ref, wih1_ref, whh1_ref, wih2_ref, whh2_ref,
                  b1_ref, b2_ref,
                  out_ref,
                  xw1_s, xw2_s, h1b_s, h1c, c1c, h2c, c2c,
                  *, t_blk, batch, hidden, out_dim, nb):
    B, H, O = batch, hidden, out_dim
    i = pl.program_id(1)

    @pl.when(i == 0)
    def _():
        h1c[...] = jnp.zeros_like(h1c)
        c1c[...] = jnp.zeros_like(c1c)
        xw2_s[...] = jnp.zeros_like(xw2_s)
        h2c[...] = jnp.zeros_like(h2c)
        c2c[...] = jnp.zeros_like(c2c)

    @pl.when(i == 1)
    def _():
        h2c[...] = jnp.zeros_like(h2c)
        c2c[...] = jnp.zeros_like(c2c)

    proj = jnp.dot(x_ref[...].reshape(t_blk * B, x_ref.shape[-1]),
                   wih1_ref[...], preferred_element_type=jnp.float32)
    xw1_s[...] = (proj + b1_ref[...]).reshape(t_blk, B, 4 * H)

    whh1 = whh1_ref[...]
    whh2 = whh2_ref[...]

    def step(k, carry):
        h1, c1, h2, c2 = carry
        g1 = jnp.dot(h1, whh1, preferred_element_type=jnp.float32) + xw1_s[k]
        g2 = jnp.dot(h2, whh2, preferred_element_type=jnp.float32) + xw2_s[k]
        h1n, c1n = _cell(g1, c1, H)
        h2n, c2n = _cell(g2, c2, O)
        h1b_s[k] = h1n.astype(jnp.bfloat16)
        out_ref[k] = h2n
        return (h1n.astype(jnp.bfloat16), c1n, h2n.astype(jnp.bfloat16), c2n)

    carry = (h1c[...], c1c[...], h2c[...], c2c[...])
    h1, c1, h2, c2 = lax.fori_loop(0, t_blk, step, carry, unroll=t_blk)
    h1c[...] = h1
    c1c[...] = c1
    h2c[...] = h2
    c2c[...] = c2

    xw2_s[...] = (jnp.dot(h1b_s[...].reshape(t_blk * B, H), wih2_ref[...],
                          preferred_element_type=jnp.float32)
                  + b2_ref[...]).reshape(t_blk, B, 4 * O)


def _stacked_lstm(x, p1, p2, *, t_blk=16):
    S, B, D = x.shape
    H = p1['w_hh'].shape[1]
    O = p2['w_hh'].shape[1]
    f32, bf16 = jnp.float32, jnp.bfloat16

    while S % t_blk:
        t_blk //= 2
    nb = S // t_blk
    ncores = 1
    Bc = B // ncores

    xb = x.astype(bf16)
    wih1 = jnp.asarray(p1['w_ih'].T, bf16)
    whh1 = jnp.asarray(p1['w_hh'].T, bf16)
    wih2 = jnp.asarray(p2['w_ih'].T, bf16)
    whh2 = jnp.asarray(p2['w_hh'].T, bf16)
    b1 = (p1['b_ih'] + p1['b_hh']).reshape(1, 4 * H).astype(f32)
    b2 = (p2['b_ih'] + p2['b_hh']).reshape(1, 4 * O).astype(f32)

    body = functools.partial(_lstm2_kernel, t_blk=t_blk, batch=Bc,
                             hidden=H, out_dim=O, nb=nb)

    x_idx = lambda c, i: (jnp.minimum(i, nb - 1), c, 0)
    o_idx = lambda c, i: (jnp.maximum(i - 1, 0), c, 0)
    const = lambda c, i: (0, 0)

    out = pl.pallas_call(
        body,
        out_shape=jax.ShapeDtypeStruct((S, B, O), f32),
        grid_spec=pltpu.PrefetchScalarGridSpec(
            num_scalar_prefetch=0,
            grid=(ncores, nb + 1),
            in_specs=[
                pl.BlockSpec((t_blk, Bc, D), x_idx),
                pl.BlockSpec((D, 4 * H), const),
                pl.BlockSpec((H, 4 * H), const),
                pl.BlockSpec((H, 4 * O), const),
                pl.BlockSpec((O, 4 * O), const),
                pl.BlockSpec((1, 4 * H), const),
                pl.BlockSpec((1, 4 * O), const),
            ],
            out_specs=pl.BlockSpec((t_blk, Bc, O), o_idx),
            scratch_shapes=[
                pltpu.VMEM((t_blk, Bc, 4 * H), f32),
                pltpu.VMEM((t_blk, Bc, 4 * O), f32),
                pltpu.VMEM((t_blk, Bc, H), bf16),
                pltpu.VMEM((Bc, H), bf16),
                pltpu.VMEM((Bc, H), f32),
                pltpu.VMEM((Bc, O), bf16),
                pltpu.VMEM((Bc, O), f32),
            ],
        ),
        compiler_params=pltpu.CompilerParams(
            dimension_semantics=("parallel", "arbitrary"),
            vmem_limit_bytes=100 * 1024 * 1024),
    )(xb, wih1, whh1, wih2, whh2, b1, b2)
    return out


def kernel(sequence, lstm_w_ih, lstm_w_hh, lstm_b_ih, lstm_b_hh,
           fc_w_ih, fc_w_hh, fc_b_ih, fc_b_hh):
    p1 = dict(w_ih=lstm_w_ih, w_hh=lstm_w_hh, b_ih=lstm_b_ih, b_hh=lstm_b_hh)
    p2 = dict(w_ih=fc_w_ih, w_hh=fc_w_hh, b_ih=fc_b_ih, b_hh=fc_b_hh)
    return _stacked_lstm(sequence, p1, p2)
